# baseline (speedup 1.0000x reference)
import functools

import jax
import jax.numpy as jnp
from jax import lax
from jax.experimental import pallas as pl
from jax.experimental.pallas import tpu as pltpu
from jax.experimental.pallas import tpu_sc as plsc

N_SPECIES = 95
N_ATOMS = 10000
N_PAIRS = 640000
S6 = 1.0
S8 = 0.2641
A2 = 5.4959
K6 = float(A2**6)
K8 = float(A2**8)

NC = 2
NS = 16
L = 16
NW = NC * NS
CHUNK = 2560
NCHUNK = N_PAIRS // CHUNK
PHASES = 2 * (-(-NCHUNK // (2 * NW)))
CSTEPS = CHUNK // L


def _body(species_hbm, aidx_hbm, dist_hbm, c6_hbm, c8_hbm,
          out_hbm, species_v, c6_v, c8_v, idx_a, idx_b, dist_a, dist_b,
          acc_v, sem_tab, sem0, sem1):
    wid = lax.axis_index("s") * NC + lax.axis_index("c")
    bufs = ((idx_a, dist_a, sem0), (idx_b, dist_b, sem1))

    def chunk_id(j):
        return jnp.minimum(wid + NW * j, NCHUNK - 1)

    def fire(j, buf):
        ib, db, sem = bufs[buf]
        off = chunk_id(j) * CHUNK
        return (
            pltpu.async_copy(aidx_hbm.at[:, pl.ds(off, CHUNK)], ib, sem),
            pltpu.async_copy(dist_hbm.at[pl.ds(off, CHUNK)], db, sem),
        )

    tab_copies = (
        pltpu.async_copy(species_hbm, species_v, sem_tab),
        pltpu.async_copy(c6_hbm, c6_v, sem_tab),
        pltpu.async_copy(c8_hbm, c8_v, sem_tab),
    )
    fire(0, 0)
    fire(1, 1)
    for cp in tab_copies:
        cp.wait()

    def phase(j, acc, buf):
        ib, db, sem = bufs[buf]
        pltpu.make_async_copy(aidx_hbm.at[:, pl.ds(0, CHUNK)], ib, sem).wait()
        pltpu.make_async_copy(dist_hbm.at[pl.ds(0, CHUNK)], db, sem).wait()

        def step(i, a):
            off = i * L
            i0 = ib[0, pl.ds(off, L)]
            i1 = ib[1, pl.ds(off, L)]
            d = db[pl.ds(off, L)]
            s0 = plsc.load_gather(species_v, [i0])
            s1 = plsc.load_gather(species_v, [i1])
            c6 = plsc.load_gather(c6_v, [s0, s1])
            c8 = plsc.load_gather(c8_v, [s0, s1])
            d2 = d * d
            d4 = d2 * d2
            d6 = d4 * d2
            d8 = d4 * d4
            den6 = d6 + K6
            den8 = d8 + K8
            num = c6 * den8 + (S8 * c8) * den6
            return a + num / (den6 * den8)

        c_acc = lax.fori_loop(0, CSTEPS, step, jnp.zeros((L,), jnp.float32))
        valid = (wid + NW * j < NCHUNK).astype(jnp.float32)
        fire(j + 2, buf)
        return acc + c_acc * jnp.broadcast_to(valid, (L,))

    def pair(g, acc):
        acc = phase(2 * g, acc, 0)
        return phase(2 * g + 1, acc, 1)

    acc = lax.fori_loop(0, PHASES // 2, pair, jnp.zeros((L,), jnp.float32))

    for buf in (0, 1):
        ib, db, sem = bufs[buf]
        pltpu.make_async_copy(aidx_hbm.at[:, pl.ds(0, CHUNK)], ib, sem).wait()
        pltpu.make_async_copy(dist_hbm.at[pl.ds(0, CHUNK)], db, sem).wait()

    acc_v[...] = acc
    pltpu.sync_copy(acc_v, out_hbm.at[wid])


_disp = functools.partial(
    pl.kernel,
    out_type=jax.ShapeDtypeStruct((NW, L), jnp.float32),
    mesh=plsc.VectorSubcoreMesh(core_axis_name="c", subcore_axis_name="s"),
    compiler_params=pltpu.CompilerParams(needs_layout_passes=False),
    scratch_types=[
        pltpu.VMEM((N_ATOMS,), jnp.int32),
        pltpu.VMEM((N_SPECIES, N_SPECIES), jnp.float32),
        pltpu.VMEM((N_SPECIES, N_SPECIES), jnp.float32),
        pltpu.VMEM((2, CHUNK), jnp.int32),
        pltpu.VMEM((2, CHUNK), jnp.int32),
        pltpu.VMEM((CHUNK,), jnp.float32),
        pltpu.VMEM((CHUNK,), jnp.float32),
        pltpu.VMEM((L,), jnp.float32),
        pltpu.SemaphoreType.DMA,
        pltpu.SemaphoreType.DMA,
        pltpu.SemaphoreType.DMA,
    ],
)(_body)


def kernel(species, energies, atom_index12, distances, c6_table, c8_table):
    partials = _disp(species.reshape(-1), atom_index12, distances,
                     c6_table, c8_table)
    return species, energies - 0.5 * jnp.sum(partials)


# speedup vs baseline: 1.0516x; 1.0516x over previous
import functools

import jax
import jax.numpy as jnp
from jax import lax
from jax.experimental import pallas as pl
from jax.experimental.pallas import tpu as pltpu
from jax.experimental.pallas import tpu_sc as plsc

N_SPECIES = 95
N_ATOMS = 10000
N_PAIRS = 640000
S6 = 1.0
S8 = 0.2641
A2 = 5.4959
K6 = float(A2**6)
K8 = float(A2**8)

NC = 2
NS = 16
L = 16
NW = NC * NS
CHUNK = 2560
NCHUNK = N_PAIRS // CHUNK
PHASES = 2 * (-(-NCHUNK // (2 * NW)))
CSTEPS = CHUNK // L


def _body(species_hbm, aidx_hbm, dist_hbm, ctab_hbm,
          out_hbm, species_v, ctab_v, idx_a, idx_b, dist_a, dist_b,
          acc_v, sem_tab, sem0, sem1):
    wid = lax.axis_index("s") * NC + lax.axis_index("c")
    bufs = ((idx_a, dist_a, sem0), (idx_b, dist_b, sem1))

    def chunk_id(j):
        return jnp.minimum(wid + NW * j, NCHUNK - 1)

    def fire(j, buf):
        ib, db, sem = bufs[buf]
        off = chunk_id(j) * CHUNK
        return (
            pltpu.async_copy(aidx_hbm.at[:, pl.ds(off, CHUNK)], ib, sem),
            pltpu.async_copy(dist_hbm.at[pl.ds(off, CHUNK)], db, sem),
        )

    tab_copies = (
        pltpu.async_copy(species_hbm, species_v, sem_tab),
        pltpu.async_copy(ctab_hbm, ctab_v, sem_tab),
    )
    fire(0, 0)
    fire(1, 1)
    for cp in tab_copies:
        cp.wait()

    def phase(j, acc, buf):
        ib, db, sem = bufs[buf]
        pltpu.make_async_copy(aidx_hbm.at[:, pl.ds(0, CHUNK)], ib, sem).wait()
        pltpu.make_async_copy(dist_hbm.at[pl.ds(0, CHUNK)], db, sem).wait()

        def step(i, a):
            off = i * L
            i0 = ib[0, pl.ds(off, L)]
            i1 = ib[1, pl.ds(off, L)]
            d = db[pl.ds(off, L)]
            s0 = plsc.load_gather(species_v, [i0])
            s1 = plsc.load_gather(species_v, [i1])
            cc = plsc.load_gather(ctab_v, [s0, s1])
            c6 = plsc.bitcast(cc << 16, jnp.float32)
            c8 = plsc.bitcast(cc & jnp.int32(-65536), jnp.float32)
            d2 = d * d
            d4 = d2 * d2
            d6 = d4 * d2
            d8 = d4 * d4
            den6 = d6 + K6
            den8 = d8 + K8
            num = c6 * den8 + (S8 * c8) * den6
            return a + num / (den6 * den8)

        c_acc = lax.fori_loop(0, CSTEPS, step, jnp.zeros((L,), jnp.float32))
        valid = (wid + NW * j < NCHUNK).astype(jnp.float32)
        fire(j + 2, buf)
        return acc + c_acc * jnp.broadcast_to(valid, (L,))

    def pair(g, acc):
        acc = phase(2 * g, acc, 0)
        return phase(2 * g + 1, acc, 1)

    acc = lax.fori_loop(0, PHASES // 2, pair, jnp.zeros((L,), jnp.float32))

    for buf in (0, 1):
        ib, db, sem = bufs[buf]
        pltpu.make_async_copy(aidx_hbm.at[:, pl.ds(0, CHUNK)], ib, sem).wait()
        pltpu.make_async_copy(dist_hbm.at[pl.ds(0, CHUNK)], db, sem).wait()

    acc_v[...] = acc
    pltpu.sync_copy(acc_v, out_hbm.at[wid])


_disp = functools.partial(
    pl.kernel,
    out_type=jax.ShapeDtypeStruct((NW, L), jnp.float32),
    mesh=plsc.VectorSubcoreMesh(core_axis_name="c", subcore_axis_name="s"),
    compiler_params=pltpu.CompilerParams(needs_layout_passes=False),
    scratch_types=[
        pltpu.VMEM((N_ATOMS,), jnp.int32),
        pltpu.VMEM((N_SPECIES, N_SPECIES), jnp.int32),
        pltpu.VMEM((2, CHUNK), jnp.int32),
        pltpu.VMEM((2, CHUNK), jnp.int32),
        pltpu.VMEM((CHUNK,), jnp.float32),
        pltpu.VMEM((CHUNK,), jnp.float32),
        pltpu.VMEM((L,), jnp.float32),
        pltpu.SemaphoreType.DMA,
        pltpu.SemaphoreType.DMA,
        pltpu.SemaphoreType.DMA,
    ],
)(_body)


def kernel(species, energies, atom_index12, distances, c6_table, c8_table):
    c6u = jax.lax.bitcast_convert_type(
        c6_table.astype(jnp.bfloat16), jnp.uint16).astype(jnp.int32)
    c8u = jax.lax.bitcast_convert_type(
        c8_table.astype(jnp.bfloat16), jnp.uint16).astype(jnp.int32)
    partials = _disp(species.reshape(-1), atom_index12, distances,
                     c6u | (c8u << 16))
    return species, energies - 0.5 * jnp.sum(partials)
